# Initial kernel scaffold; baseline (speedup 1.0000x reference)
import jax
import jax.numpy as jnp
from jax.experimental import pallas as pl


def kernel(X, edge_index, edge_weight, fn_W1, fn_b1, fn_W2, fn_b2, gat1_W, gat1_as, gat1_ad, gat1_b, gat2_W, gat2_as, gat2_ad, gat2_b, gc_W, gc_b, gen_W, gen_b, dec_W, dec_b):
    raise NotImplementedError("write your pallas kernel here")


# speedup vs baseline: 1.0016x; 1.0016x over previous
import functools

import jax
import jax.numpy as jnp
from jax.experimental import pallas as pl


N = 10000
E = 320000
IN_DIM = 128
H_DIM = 128
Z_DIM = 64

_BLK = 1000


def _gelu(x):
    return 0.5 * x * (1.0 + jax.lax.erf(x * 0.7071067811865476))


def _mlp_body(x_ref, w1_ref, b1_ref, w2_ref, b2_ref, o_ref):
    z = _gelu(jnp.dot(x_ref[...], w1_ref[...],
                      preferred_element_type=jnp.float32) + b1_ref[...])
    z = _gelu(jnp.dot(z, w2_ref[...],
                      preferred_element_type=jnp.float32) + b2_ref[...])
    o_ref[...] = z


def _mlp(X, W1, b1, W2, b2):
    grid = (N // _BLK,)
    return pl.pallas_call(
        _mlp_body,
        grid=grid,
        in_specs=[
            pl.BlockSpec((_BLK, IN_DIM), lambda i: (i, 0)),
            pl.BlockSpec((IN_DIM, H_DIM), lambda i: (0, 0)),
            pl.BlockSpec((H_DIM,), lambda i: (0,)),
            pl.BlockSpec((H_DIM, H_DIM), lambda i: (0, 0)),
            pl.BlockSpec((H_DIM,), lambda i: (0,)),
        ],
        out_specs=pl.BlockSpec((_BLK, H_DIM), lambda i: (i, 0)),
        out_shape=jax.ShapeDtypeStruct((N, H_DIM), jnp.float32),
    )(X, W1, b1, W2, b2)


def _gat(x, edge_index, W, att_src, att_dst, bias, num_nodes):
    loop = jnp.arange(num_nodes, dtype=edge_index.dtype)
    src = jnp.concatenate([edge_index[0], loop])
    dst = jnp.concatenate([edge_index[1], loop])
    h = x @ W
    a_src = jnp.sum(h * att_src, axis=-1)
    a_dst = jnp.sum(h * att_dst, axis=-1)
    alpha = a_src[src] + a_dst[dst]
    alpha = jax.nn.leaky_relu(alpha, negative_slope=0.2)
    amax = jax.ops.segment_max(alpha, dst, num_segments=num_nodes)
    alpha = jnp.exp(alpha - amax[dst])
    denom = jax.ops.segment_sum(alpha, dst, num_segments=num_nodes)
    alpha = alpha / (denom[dst] + 1e-16)
    out = jax.ops.segment_sum(h[src] * alpha[:, None], dst,
                              num_segments=num_nodes)
    return out + bias


def kernel(X, edge_index, edge_weight, fn_W1, fn_b1, fn_W2, fn_b2,
           gat1_W, gat1_as, gat1_ad, gat1_b,
           gat2_W, gat2_as, gat2_ad, gat2_b,
           gc_W, gc_b, gen_W, gen_b, dec_W, dec_b):
    num_nodes = X.shape[0]
    z = _mlp(X, fn_W1, fn_b1, fn_W2, fn_b2)
    z = _gat(z, edge_index, gat1_W, gat1_as, gat1_ad, gat1_b, num_nodes)
    z = _gelu(z)
    z = _gat(z, edge_index, gat2_W, gat2_as, gat2_ad, gat2_b, num_nodes)
    z = _gelu(z)
    z = z @ gc_W + gc_b
    z = z @ gen_W + gen_b
    X_hat = z @ dec_W + dec_b
    return jnp.mean((X_hat - X) ** 2)


# speedup vs baseline: 29.5958x; 29.5494x over previous
import jax
import jax.numpy as jnp
from jax import lax
from jax.experimental import pallas as pl
from jax.experimental.pallas import tpu as pltpu
from jax.experimental.pallas import tpu_sc as plsc

N = 10000
IN_DIM = 128
H_DIM = 128
Z_DIM = 64

NP = 10112
E = 320000
EA = E + N
CH = 128
NWORK = 32
CPT = 81
EP = NWORK * CPT * CH
RPT = NP // 16

_BLK = 1024


def _gelu(x):
    return 0.5 * x * (1.0 + jax.lax.erf(x * 0.7071067811865476))


def _tc1_body(x_ref, w1_ref, b1_ref, w2_ref, b2_ref, gw_ref, as_ref, ad_ref,
              h_ref, s_ref, d_ref):
    z = _gelu(jnp.dot(x_ref[...], w1_ref[...],
                      preferred_element_type=jnp.float32) + b1_ref[...])
    z = _gelu(jnp.dot(z, w2_ref[...],
                      preferred_element_type=jnp.float32) + b2_ref[...])
    h = jnp.dot(z, gw_ref[...], preferred_element_type=jnp.float32)
    h_ref[...] = h
    s_ref[...] = jnp.sum(h * as_ref[...], axis=1)
    d_ref[...] = jnp.sum(h * ad_ref[...], axis=1)


def _tc1(X, W1, b1, W2, b2, gW, a_s, a_d):
    grid = ((N + _BLK - 1) // _BLK,)
    return pl.pallas_call(
        _tc1_body,
        grid=grid,
        in_specs=[
            pl.BlockSpec((_BLK, IN_DIM), lambda i: (i, 0)),
            pl.BlockSpec((IN_DIM, H_DIM), lambda i: (0, 0)),
            pl.BlockSpec((H_DIM,), lambda i: (0,)),
            pl.BlockSpec((H_DIM, H_DIM), lambda i: (0, 0)),
            pl.BlockSpec((H_DIM,), lambda i: (0,)),
            pl.BlockSpec((H_DIM, Z_DIM), lambda i: (0, 0)),
            pl.BlockSpec((1, Z_DIM), lambda i: (0, 0)),
            pl.BlockSpec((1, Z_DIM), lambda i: (0, 0)),
        ],
        out_specs=[
            pl.BlockSpec((_BLK, Z_DIM), lambda i: (i, 0)),
            pl.BlockSpec((_BLK,), lambda i: (i,)),
            pl.BlockSpec((_BLK,), lambda i: (i,)),
        ],
        out_shape=[
            jax.ShapeDtypeStruct((N, Z_DIM), jnp.float32),
            jax.ShapeDtypeStruct((N,), jnp.float32),
            jax.ShapeDtypeStruct((N,), jnp.float32),
        ],
    )(X, W1, b1, W2, b2, gW, a_s.reshape(1, Z_DIM), a_d.reshape(1, Z_DIM))


def _sc_gat_body(src_h, dst_h, asrc_h, adst_h, shift_h, hp_h,
                 out_h, den_h,
                 src_v, dst_v, asrc_v, adst_v, shift_v, hrows, erows,
                 out_sp, den_sp, sem):
    cid = lax.axis_index("c")
    sid = lax.axis_index("s")
    tilebase = cid * 16 + sid
    rowbase = sid * RPT

    pltpu.sync_copy(asrc_h, asrc_v)
    pltpu.sync_copy(adst_h, adst_v)
    pltpu.sync_copy(shift_h, shift_v)
    pltpu.sync_copy(src_h.at[tilebase], src_v)
    pltpu.sync_copy(dst_h.at[tilebase], dst_v)

    def zrow(r, carry):
        for g in range(Z_DIM // 16):
            hrows[r, pl.ds(g * 16, 16)] = jnp.zeros((16,), jnp.float32)
        erows[r, :] = jnp.zeros((16,), jnp.float32)
        return carry
    lax.fori_loop(0, CH, zrow, 0)

    for off, sz in ((0, 128), (128, 128), (256, 128), (384, 128), (512, 120)):
        pltpu.sync_copy(hrows.at[pl.ds(0, sz)],
                        out_sp.at[pl.ds(rowbase + off, sz)])
        pltpu.sync_copy(erows.at[pl.ds(0, sz)],
                        den_sp.at[pl.ds(rowbase + off, sz)])

    plsc.subcore_barrier()

    zero16 = jnp.zeros((16,), jnp.int32)
    lane = lax.iota(jnp.int32, 16)

    def chunk_body(ck, carry):
        cp = pltpu.async_copy(hp_h.at[src_v.at[ck]], hrows, sem)

        def e_body(j, c2):
            s16 = src_v[ck, pl.ds(j * 16, 16)]
            d16 = dst_v[ck, pl.ds(j * 16, 16)]
            a = plsc.load_gather(asrc_v, [s16]) + plsc.load_gather(adst_v, [d16])
            a = jnp.maximum(a, 0.2 * a)
            e16 = jnp.exp(a - plsc.load_gather(shift_v, [d16]))
            plsc.store_scatter(erows, [j * 16 + lane, zero16], e16)
            return c2
        lax.fori_loop(0, CH // 16, e_body, 0)

        cp.wait()

        def m_body(r, c2):
            er = plsc.load_gather(erows, [jnp.full((16,), r, jnp.int32), zero16])
            for g in range(Z_DIM // 16):
                hrows[r, pl.ds(g * 16, 16)] = hrows[r, pl.ds(g * 16, 16)] * er
            return c2
        lax.fori_loop(0, CH, m_body, 0)

        pltpu.sync_copy(hrows, out_sp.at[dst_v.at[ck]], add=True)
        pltpu.sync_copy(erows, den_sp.at[dst_v.at[ck]], add=True)
        return carry
    lax.fori_loop(0, CPT, chunk_body, 0)

    plsc.subcore_barrier()
    pltpu.sync_copy(out_sp.at[pl.ds(rowbase, RPT)],
                    out_h.at[cid].at[pl.ds(rowbase, RPT)])
    pltpu.sync_copy(den_sp.at[pl.ds(rowbase, RPT)],
                    den_h.at[cid].at[pl.ds(rowbase, RPT)])


def _sc_gat(src2d, dst2d, asrc, adst, shift, hp):
    f = pl.kernel(
        _sc_gat_body,
        out_type=(jax.ShapeDtypeStruct((2, NP, Z_DIM), jnp.float32),
                  jax.ShapeDtypeStruct((2, NP, 16), jnp.float32)),
        mesh=plsc.VectorSubcoreMesh(core_axis_name="c", subcore_axis_name="s"),
        compiler_params=pltpu.CompilerParams(needs_layout_passes=False,
                                             use_tc_tiling_on_sc=False),
        scratch_types=[
            pltpu.VMEM((CPT, CH), jnp.int32),
            pltpu.VMEM((CPT, CH), jnp.int32),
            pltpu.VMEM((NP,), jnp.float32),
            pltpu.VMEM((NP,), jnp.float32),
            pltpu.VMEM((NP,), jnp.float32),
            pltpu.VMEM((CH, Z_DIM), jnp.float32),
            pltpu.VMEM((CH, 16), jnp.float32),
            pltpu.VMEM_SHARED((NP, Z_DIM), jnp.float32),
            pltpu.VMEM_SHARED((NP, 16), jnp.float32),
            pltpu.SemaphoreType.DMA,
        ],
    )
    return f(src2d, dst2d, asrc, adst, shift, hp)


def _tc2_body(out_ref, den_ref, b_ref, w_ref, as_ref, ad_ref,
              h_ref, s_ref, d_ref):
    agg = out_ref[0] + out_ref[1]
    den = den_ref[0, :, 0:1] + den_ref[1, :, 0:1]
    x = _gelu(agg / (den + 1e-16) + b_ref[...])
    h = jnp.dot(x, w_ref[...], preferred_element_type=jnp.float32)
    h_ref[...] = h
    s_ref[...] = jnp.sum(h * as_ref[...], axis=1)
    d_ref[...] = jnp.sum(h * ad_ref[...], axis=1)


def _tc2(out1, den1, b, W, a_s, a_d):
    grid = ((N + _BLK - 1) // _BLK,)
    return pl.pallas_call(
        _tc2_body,
        grid=grid,
        in_specs=[
            pl.BlockSpec((2, _BLK, Z_DIM), lambda i: (0, i, 0)),
            pl.BlockSpec((2, _BLK, 16), lambda i: (0, i, 0)),
            pl.BlockSpec((1, Z_DIM), lambda i: (0, 0)),
            pl.BlockSpec((Z_DIM, Z_DIM), lambda i: (0, 0)),
            pl.BlockSpec((1, Z_DIM), lambda i: (0, 0)),
            pl.BlockSpec((1, Z_DIM), lambda i: (0, 0)),
        ],
        out_specs=[
            pl.BlockSpec((_BLK, Z_DIM), lambda i: (i, 0)),
            pl.BlockSpec((_BLK,), lambda i: (i,)),
            pl.BlockSpec((_BLK,), lambda i: (i,)),
        ],
        out_shape=[
            jax.ShapeDtypeStruct((N, Z_DIM), jnp.float32),
            jax.ShapeDtypeStruct((N,), jnp.float32),
            jax.ShapeDtypeStruct((N,), jnp.float32),
        ],
    )(out1, den1, b.reshape(1, Z_DIM), W,
      a_s.reshape(1, Z_DIM), a_d.reshape(1, Z_DIM))


def _tc3_body(out_ref, den_ref, b_ref, gcw_ref, gcb_ref, genw_ref, genb_ref,
              decw_ref, decb_ref, x_ref, acc_ref):
    i = pl.program_id(0)
    agg = out_ref[0] + out_ref[1]
    den = den_ref[0, :, 0:1] + den_ref[1, :, 0:1]
    z = _gelu(agg / (den + 1e-16) + b_ref[...])
    z = jnp.dot(z, gcw_ref[...], preferred_element_type=jnp.float32) + gcb_ref[...]
    z = jnp.dot(z, genw_ref[...], preferred_element_type=jnp.float32) + genb_ref[...]
    xh = jnp.dot(z, decw_ref[...], preferred_element_type=jnp.float32) + decb_ref[...]
    d = xh - x_ref[...]
    rows = i * _BLK + lax.broadcasted_iota(jnp.int32, (_BLK, IN_DIM), 0)
    d = jnp.where(rows < N, d, 0.0)
    s = jnp.sum(d * d).reshape(1, 1)

    @pl.when(i == 0)
    def _():
        acc_ref[...] = jnp.zeros((1, 1), jnp.float32)
    acc_ref[...] += s


def _tc3(out2, den2, b, gcW, gcb, genW, genb, decW, decb, X):
    grid = ((N + _BLK - 1) // _BLK,)
    return pl.pallas_call(
        _tc3_body,
        grid=grid,
        in_specs=[
            pl.BlockSpec((2, _BLK, Z_DIM), lambda i: (0, i, 0)),
            pl.BlockSpec((2, _BLK, 16), lambda i: (0, i, 0)),
            pl.BlockSpec((1, Z_DIM), lambda i: (0, 0)),
            pl.BlockSpec((Z_DIM, Z_DIM), lambda i: (0, 0)),
            pl.BlockSpec((1, Z_DIM), lambda i: (0, 0)),
            pl.BlockSpec((Z_DIM, Z_DIM), lambda i: (0, 0)),
            pl.BlockSpec((1, Z_DIM), lambda i: (0, 0)),
            pl.BlockSpec((Z_DIM, IN_DIM), lambda i: (0, 0)),
            pl.BlockSpec((1, IN_DIM), lambda i: (0, 0)),
            pl.BlockSpec((_BLK, IN_DIM), lambda i: (i, 0)),
        ],
        out_specs=pl.BlockSpec((1, 1), lambda i: (0, 0)),
        out_shape=jax.ShapeDtypeStruct((1, 1), jnp.float32),
    )(out2, den2, b.reshape(1, Z_DIM), gcW, gcb.reshape(1, Z_DIM),
      genW, genb.reshape(1, Z_DIM), decW, decb.reshape(1, IN_DIM), X)


def _shift_and_pad(a_s, a_d):
    t = jnp.max(a_s) + a_d
    shift = jnp.maximum(t, 0.2 * t)
    pad = NP - N
    return (jnp.pad(a_s, (0, pad)), jnp.pad(a_d, (0, pad)),
            jnp.pad(shift, (0, pad)))


def kernel(X, edge_index, edge_weight, fn_W1, fn_b1, fn_W2, fn_b2,
           gat1_W, gat1_as, gat1_ad, gat1_b,
           gat2_W, gat2_as, gat2_ad, gat2_b,
           gc_W, gc_b, gen_W, gen_b, dec_W, dec_b):
    loop = jnp.arange(N, dtype=jnp.int32)
    padi = jnp.full((EP - EA,), N, jnp.int32)
    src2d = jnp.concatenate([edge_index[0], loop, padi]).reshape(NWORK, CPT, CH)
    dst2d = jnp.concatenate([edge_index[1], loop, padi]).reshape(NWORK, CPT, CH)
    pad = NP - N

    h1, as1, ad1 = _tc1(X, fn_W1, fn_b1, fn_W2, fn_b2, gat1_W, gat1_as, gat1_ad)
    asrc1, adst1, shift1 = _shift_and_pad(as1, ad1)
    hp1 = jnp.pad(h1, ((0, pad), (0, 0)))
    out1, den1 = _sc_gat(src2d, dst2d, asrc1, adst1, shift1, hp1)

    h2, as2, ad2 = _tc2(out1, den1, gat1_b, gat2_W, gat2_as, gat2_ad)
    asrc2, adst2, shift2 = _shift_and_pad(as2, ad2)
    hp2 = jnp.pad(h2, ((0, pad), (0, 0)))
    out2, den2 = _sc_gat(src2d, dst2d, asrc2, adst2, shift2, hp2)

    acc = _tc3(out2, den2, gat2_b, gc_W, gc_b, gen_W, gen_b, dec_W, dec_b, X)
    return acc[0, 0] / float(N * IN_DIM)
